# baseline (speedup 1.0000x reference)
import functools

import jax
import jax.numpy as jnp
from jax import lax
from jax.experimental import pallas as pl
from jax.experimental.pallas import tpu as pltpu
from jax.experimental.pallas import tpu_sc as plsc

N = 10000
D = 128
E = 320000
NS = 16
NB = 128
GRP = 8
NGRP = 20
NCHUNK = NGRP * GRP
EPT = NCHUNK * NB
EP = EPT * NS
ACC_ROWS = 10112
DUMMY = N
NZCH = ACC_ROWS // NB
FROWS = 624

_mesh = plsc.VectorSubcoreMesh(core_axis_name="c", subcore_axis_name="s")


@functools.partial(
    pl.kernel,
    mesh=_mesh,
    out_type=jax.ShapeDtypeStruct((2, 2, N, D), jnp.float32),
    scratch_types=[
        pltpu.VMEM((3 * GRP, NB), jnp.int32),
        pltpu.VMEM((3 * GRP, NB), jnp.int32),
        pltpu.VMEM((2, NB, D), jnp.float32),
        pltpu.VMEM_SHARED((ACC_ROWS, D), jnp.float32),
        pltpu.SemaphoreType.DMA,
        pltpu.SemaphoreType.DMA,
        pltpu.SemaphoreType.DMA,
        pltpu.SemaphoreType.DMA,
        pltpu.SemaphoreType.DMA,
    ],
)
def _sc_scatter(x_a_hbm, x_b_hbm, src_hbm, dst_hbm, out_hbm,
                src_v, dst_v, rows_v, accum, gsem0, gsem1, ssem0, ssem1, isem):
    cid = lax.axis_index("c")
    tid = lax.axis_index("s")
    gsem = (gsem0, gsem1)
    ssem = (ssem0, ssem1)

    def _stage_idx(p, g, buf):
        pltpu.async_copy(src_hbm.at[p, cid, tid, pl.ds(g * GRP, GRP)],
                         src_v.at[pl.ds(buf * GRP, GRP)], isem)
        pltpu.async_copy(dst_hbm.at[p, cid, tid, pl.ds(g * GRP, GRP)],
                         dst_v.at[pl.ds(buf * GRP, GRP)], isem)

    def _wait_idx(p, g, buf):
        pltpu.make_async_copy(src_hbm.at[p, cid, tid, pl.ds(g * GRP, GRP)],
                              src_v.at[pl.ds(buf * GRP, GRP)], isem).wait()
        pltpu.make_async_copy(dst_hbm.at[p, cid, tid, pl.ds(g * GRP, GRP)],
                              dst_v.at[pl.ds(buf * GRP, GRP)], isem).wait()

    for p in range(2):
        x_hbm = x_a_hbm if p == 0 else x_b_hbm

        def _zrow(i, carry):
            for j in range(8):
                rows_v[0, i, pl.ds(j * 16, 16)] = jnp.zeros((16,), jnp.float32)
            return carry

        lax.fori_loop(0, NB, _zrow, 0)
        for k in range((NZCH + NS - 1) // NS):
            c = tid + k * NS
            if (k + 1) * NS <= NZCH:
                pltpu.sync_copy(rows_v.at[0], accum.at[pl.ds(c * NB, NB)])
            else:
                @pl.when(c < NZCH)
                def _ztail():
                    pltpu.sync_copy(rows_v.at[0], accum.at[pl.ds(c * NB, NB)])
        plsc.subcore_barrier()

        pltpu.sync_copy(src_hbm.at[p, cid, tid, pl.ds(0, GRP)],
                        src_v.at[pl.ds(0, GRP)])
        pltpu.sync_copy(dst_hbm.at[p, cid, tid, pl.ds(0, GRP)],
                        dst_v.at[pl.ds(0, GRP)])
        _stage_idx(p, 1, 1)
        pltpu.async_copy(x_hbm.at[src_v.at[0]], rows_v.at[0], gsem0)

        def _group(g, carry):
            base = lax.rem(g, 3) * GRP
            nxt = lax.rem(g + 1, 3) * GRP
            stage = lax.rem(g + 2, 3)
            for j in range(GRP):
                b = j % 2
                r = base + j
                pltpu.make_async_copy(x_hbm.at[src_v.at[r]], rows_v.at[b],
                                      gsem[b]).wait()
                pltpu.async_copy(rows_v.at[b], accum.at[dst_v.at[r]], ssem[b],
                                 add=True)
                if j == 0:
                    @pl.when(g > 0)
                    def _w0():
                        pltpu.make_async_copy(rows_v.at[1],
                                              accum.at[dst_v.at[r]],
                                              ssem[1]).wait()
                    pltpu.async_copy(x_hbm.at[src_v.at[r + 1]], rows_v.at[1],
                                     gsem[1])
                elif j < GRP - 1:
                    pltpu.make_async_copy(rows_v.at[1 - b],
                                          accum.at[dst_v.at[r]],
                                          ssem[1 - b]).wait()
                    pltpu.async_copy(x_hbm.at[src_v.at[r + 1]],
                                     rows_v.at[1 - b], gsem[1 - b])
                else:
                    pltpu.make_async_copy(rows_v.at[0],
                                          accum.at[dst_v.at[r]],
                                          ssem[0]).wait()

                    @pl.when(g + 1 < NGRP)
                    def _next_group():
                        _wait_idx(p, g + 1, lax.rem(g + 1, 3))
                        pltpu.async_copy(x_hbm.at[src_v.at[nxt]],
                                         rows_v.at[0], gsem[0])
                        @pl.when(g + 2 < NGRP)
                        def _stage_ahead():
                            _stage_idx(p, g + 2, stage)
            return carry

        lax.fori_loop(0, NGRP, _group, 0)
        pltpu.make_async_copy(rows_v.at[1], accum.at[dst_v.at[0]], ssem1).wait()
        plsc.subcore_barrier()
        pltpu.sync_copy(accum.at[pl.ds(tid * FROWS, FROWS)],
                        out_hbm.at[p, cid, pl.ds(tid * FROWS, FROWS)])

        @pl.when(tid == NS - 1)
        def _tail():
            pltpu.sync_copy(accum.at[pl.ds(NS * FROWS, N - NS * FROWS)],
                            out_hbm.at[p, cid, pl.ds(NS * FROWS, N - NS * FROWS)])

        plsc.subcore_barrier()


def _mm_body(a0_ref, a1_ref, w0_ref, w1_ref, o_ref):
    acc = jnp.dot(a0_ref[...], w0_ref[...],
                  preferred_element_type=jnp.float32,
                  precision=lax.Precision.HIGHEST)
    acc += jnp.dot(a1_ref[...], w1_ref[...],
                   preferred_element_type=jnp.float32,
                   precision=lax.Precision.HIGHEST)
    o_ref[...] = acc


def _combine(a0, a1, w0, w1):
    blk = 1000
    return pl.pallas_call(
        _mm_body,
        grid=(N // blk,),
        in_specs=[
            pl.BlockSpec((blk, D), lambda i: (i, 0)),
            pl.BlockSpec((blk, D), lambda i: (i, 0)),
            pl.BlockSpec((D, D), lambda i: (0, 0)),
            pl.BlockSpec((D, D), lambda i: (0, 0)),
        ],
        out_specs=pl.BlockSpec((blk, D), lambda i: (i, 0)),
        out_shape=jax.ShapeDtypeStruct((N, D), jnp.float32),
    )(a0, a1, w0, w1)


def _prep(edge_index):
    pad = EP - E
    perm = jnp.argsort(edge_index[1] // 2560, stable=True)
    src = jnp.concatenate([edge_index[0][perm], jnp.zeros((pad,), jnp.int32)])
    dst = jnp.concatenate([edge_index[1][perm],
                           jnp.full((pad,), DUMMY, jnp.int32)])
    return src.reshape(NS, NCHUNK, NB), dst.reshape(NS, NCHUNK, NB)


def kernel(x_a, x_b, edge_index_aa, edge_index_ab, edge_index_ba,
           edge_index_bb, W_aa, W_ab, W_ba, W_bb):
    s_aa, d_aa = _prep(edge_index_aa)
    s_ab, d_ab = _prep(edge_index_ab)
    s_ba, d_ba = _prep(edge_index_ba)
    s_bb, d_bb = _prep(edge_index_bb)
    src_all = jnp.stack([jnp.stack([s_aa, s_ab]), jnp.stack([s_ba, s_bb])])
    dst_all = jnp.stack([jnp.stack([d_aa, d_ab]), jnp.stack([d_ba, d_bb])])
    A = _sc_scatter(x_a, x_b, src_all, dst_all)
    out_a = _combine(A[0, 0], A[1, 0], W_aa, W_ba)
    out_b = _combine(A[0, 1], A[1, 1], W_ab, W_bb)
    return out_a, out_b


# speedup vs baseline: 1.9784x; 1.9784x over previous
import functools

import jax
import jax.numpy as jnp
from jax import lax
from jax.experimental import pallas as pl
from jax.experimental.pallas import tpu as pltpu
from jax.experimental.pallas import tpu_sc as plsc

N = 10000
D = 128
E = 320000
NS = 16
NB = 128
GRP = 8
NGRP = 20
NCHUNK = NGRP * GRP
EPT = NCHUNK * NB
EP = EPT * NS
ACC_ROWS = 10112
DUMMY = N
NZCH = ACC_ROWS // NB
FROWS = 624

_mesh = plsc.VectorSubcoreMesh(core_axis_name="c", subcore_axis_name="s")


@functools.partial(
    pl.kernel,
    mesh=_mesh,
    out_type=jax.ShapeDtypeStruct((2, 2, N, D), jnp.float32),
    scratch_types=[
        pltpu.VMEM((3 * GRP, NB), jnp.int32),
        pltpu.VMEM((3 * GRP, NB), jnp.int32),
        pltpu.VMEM((2, NB, D), jnp.float32),
        pltpu.VMEM_SHARED((ACC_ROWS, D), jnp.float32),
        pltpu.SemaphoreType.DMA,
        pltpu.SemaphoreType.DMA,
        pltpu.SemaphoreType.DMA,
        pltpu.SemaphoreType.DMA,
        pltpu.SemaphoreType.DMA,
    ],
)
def _sc_scatter(x_a_hbm, x_b_hbm, src_hbm, dst_hbm, out_hbm,
                src_v, dst_v, rows_v, accum, gsem0, gsem1, ssem0, ssem1, isem):
    cid = lax.axis_index("c")
    tid = lax.axis_index("s")
    gsem = (gsem0, gsem1)
    ssem = (ssem0, ssem1)

    def _stage_idx(p, g, buf):
        pltpu.async_copy(src_hbm.at[p, cid, tid, pl.ds(g * GRP, GRP)],
                         src_v.at[pl.ds(buf * GRP, GRP)], isem)
        pltpu.async_copy(dst_hbm.at[p, cid, tid, pl.ds(g * GRP, GRP)],
                         dst_v.at[pl.ds(buf * GRP, GRP)], isem)

    def _wait_idx(p, g, buf):
        pltpu.make_async_copy(src_hbm.at[p, cid, tid, pl.ds(g * GRP, GRP)],
                              src_v.at[pl.ds(buf * GRP, GRP)], isem).wait()
        pltpu.make_async_copy(dst_hbm.at[p, cid, tid, pl.ds(g * GRP, GRP)],
                              dst_v.at[pl.ds(buf * GRP, GRP)], isem).wait()

    for p in range(2):
        x_hbm = x_a_hbm if p == 0 else x_b_hbm

        def _zrow(i, carry):
            for j in range(8):
                rows_v[0, i, pl.ds(j * 16, 16)] = jnp.zeros((16,), jnp.float32)
            return carry

        lax.fori_loop(0, NB, _zrow, 0)
        for k in range((NZCH + NS - 1) // NS):
            c = tid + k * NS
            if (k + 1) * NS <= NZCH:
                pltpu.sync_copy(rows_v.at[0], accum.at[pl.ds(c * NB, NB)])
            else:
                @pl.when(c < NZCH)
                def _ztail():
                    pltpu.sync_copy(rows_v.at[0], accum.at[pl.ds(c * NB, NB)])
        plsc.subcore_barrier()

        pltpu.sync_copy(src_hbm.at[p, cid, tid, pl.ds(0, GRP)],
                        src_v.at[pl.ds(0, GRP)])
        pltpu.sync_copy(dst_hbm.at[p, cid, tid, pl.ds(0, GRP)],
                        dst_v.at[pl.ds(0, GRP)])
        _stage_idx(p, 1, 1)
        pltpu.async_copy(x_hbm.at[src_v.at[0]], rows_v.at[0], gsem0)

        def _group(g, carry):
            base = lax.rem(g, 3) * GRP
            nxt = lax.rem(g + 1, 3) * GRP
            stage = lax.rem(g + 2, 3)
            for j in range(GRP):
                b = j % 2
                r = base + j
                pltpu.make_async_copy(x_hbm.at[src_v.at[r]], rows_v.at[b],
                                      gsem[b]).wait()
                pltpu.async_copy(rows_v.at[b], accum.at[dst_v.at[r]], ssem[b],
                                 add=True)
                if j == 0:
                    @pl.when(g > 0)
                    def _w0():
                        pltpu.make_async_copy(rows_v.at[1],
                                              accum.at[dst_v.at[r]],
                                              ssem[1]).wait()
                    pltpu.async_copy(x_hbm.at[src_v.at[r + 1]], rows_v.at[1],
                                     gsem[1])
                elif j < GRP - 1:
                    pltpu.make_async_copy(rows_v.at[1 - b],
                                          accum.at[dst_v.at[r]],
                                          ssem[1 - b]).wait()
                    pltpu.async_copy(x_hbm.at[src_v.at[r + 1]],
                                     rows_v.at[1 - b], gsem[1 - b])
                else:
                    pltpu.make_async_copy(rows_v.at[0],
                                          accum.at[dst_v.at[r]],
                                          ssem[0]).wait()

                    @pl.when(g + 1 < NGRP)
                    def _next_group():
                        _wait_idx(p, g + 1, lax.rem(g + 1, 3))
                        pltpu.async_copy(x_hbm.at[src_v.at[nxt]],
                                         rows_v.at[0], gsem[0])
                        @pl.when(g + 2 < NGRP)
                        def _stage_ahead():
                            _stage_idx(p, g + 2, stage)
            return carry

        lax.fori_loop(0, NGRP, _group, 0)
        pltpu.make_async_copy(rows_v.at[1], accum.at[dst_v.at[0]], ssem1).wait()
        plsc.subcore_barrier()
        pltpu.sync_copy(accum.at[pl.ds(tid * FROWS, FROWS)],
                        out_hbm.at[p, cid, pl.ds(tid * FROWS, FROWS)])

        @pl.when(tid == NS - 1)
        def _tail():
            pltpu.sync_copy(accum.at[pl.ds(NS * FROWS, N - NS * FROWS)],
                            out_hbm.at[p, cid, pl.ds(NS * FROWS, N - NS * FROWS)])

        plsc.subcore_barrier()


def _mm_body(a0_ref, a1_ref, w0_ref, w1_ref, o_ref):
    acc = jnp.dot(a0_ref[0, 0], w0_ref[...],
                  preferred_element_type=jnp.float32,
                  precision=lax.Precision.HIGHEST)
    acc += jnp.dot(a1_ref[0, 0], w1_ref[...],
                   preferred_element_type=jnp.float32,
                   precision=lax.Precision.HIGHEST)
    o_ref[...] = acc


def _combine(A, c, w0, w1):
    blk = 1000
    return pl.pallas_call(
        _mm_body,
        grid=(N // blk,),
        in_specs=[
            pl.BlockSpec((1, 1, blk, D), lambda i: (0, c, i, 0)),
            pl.BlockSpec((1, 1, blk, D), lambda i: (1, c, i, 0)),
            pl.BlockSpec((D, D), lambda i: (0, 0)),
            pl.BlockSpec((D, D), lambda i: (0, 0)),
        ],
        out_specs=pl.BlockSpec((blk, D), lambda i: (i, 0)),
        out_shape=jax.ShapeDtypeStruct((N, D), jnp.float32),
    )(A, A, w0, w1)


def _prep(edge_index):
    pad = EP - E
    src = jnp.concatenate([edge_index[0], jnp.zeros((pad,), jnp.int32)])
    dst = jnp.concatenate([edge_index[1], jnp.full((pad,), DUMMY, jnp.int32)])
    return src.reshape(NS, NCHUNK, NB), dst.reshape(NS, NCHUNK, NB)


def kernel(x_a, x_b, edge_index_aa, edge_index_ab, edge_index_ba,
           edge_index_bb, W_aa, W_ab, W_ba, W_bb):
    s_aa, d_aa = _prep(edge_index_aa)
    s_ab, d_ab = _prep(edge_index_ab)
    s_ba, d_ba = _prep(edge_index_ba)
    s_bb, d_bb = _prep(edge_index_bb)
    src_all = jnp.stack([jnp.stack([s_aa, s_ab]), jnp.stack([s_ba, s_bb])])
    dst_all = jnp.stack([jnp.stack([d_aa, d_ab]), jnp.stack([d_ba, d_bb])])
    A = _sc_scatter(x_a, x_b, src_all, dst_all)
    out_a = _combine(A, 0, W_aa, W_ba)
    out_b = _combine(A, 1, W_ab, W_bb)
    return out_a, out_b
